# baseline (speedup 1.0000x reference)
import functools

import jax
import jax.numpy as jnp
from jax import lax
from jax.experimental import pallas as pl
from jax.experimental.pallas import tpu as pltpu
from jax.experimental.pallas import tpu_sc as plsc

_BATCH = 4096
_HIST = 200
_HISTP = 256
_RING = 208
_VOCAB = 1000000
_DIM = 64
_NCLASS = 100
_PROJ = 128
_PKW = 64

_NCHUNK = 13
_CHUNK = 16

_MM_BLK = 2048


def _tc_project(table_t, fc_weight):
  def body(t_ref, w_ref, o_ref):
    acc = lax.dot_general(
        t_ref[:, :], w_ref[:, :],
        dimension_numbers=(((0,), (1,)), ((), ())),
        preferred_element_type=jnp.float32,
    )
    p = jnp.pad(acc, ((0, 0), (0, _PROJ - _NCLASS))).astype(jnp.bfloat16)
    lo = lax.bitcast_convert_type(p[:, :_PKW], jnp.uint16).astype(jnp.uint32)
    hi = lax.bitcast_convert_type(p[:, _PKW:], jnp.uint16).astype(jnp.uint32)
    packed = lo | (hi << 16)
    o_ref[:, :] = jnp.concatenate(
        [packed[:_MM_BLK // 2, :], packed[_MM_BLK // 2:, :]], axis=1)

  return pl.pallas_call(
      body,
      grid=((_VOCAB + _MM_BLK - 1) // _MM_BLK,),
      in_specs=[
          pl.BlockSpec((_DIM, _MM_BLK), lambda i: (0, i)),
          pl.BlockSpec((_NCLASS, _DIM), lambda i: (0, 0)),
      ],
      out_specs=pl.BlockSpec((_MM_BLK // 2, 2 * _PKW), lambda i: (i, 0)),
      out_shape=jax.ShapeDtypeStruct(
          (((_VOCAB + _MM_BLK - 1) // _MM_BLK) * (_MM_BLK // 2), 2 * _PKW),
          jnp.uint32),
      compiler_params=pltpu.CompilerParams(fuse_transposed_lhs_in_matmul=True),
  )(table_t, fc_weight)


def _sc_gather_sum(pair2, hoff2, proj):
  mesh = plsc.VectorSubcoreMesh(core_axis_name="c", subcore_axis_name="s")
  nw = mesh.num_cores * mesh.num_subcores
  rows_per_w = _BATCH // nw
  half = rows_per_w // 2

  @functools.partial(
      pl.kernel,
      out_type=jax.ShapeDtypeStruct((_BATCH, _PROJ), jnp.float32),
      mesh=mesh,
      scratch_types=[
          pltpu.VMEM((half, _HISTP), jnp.int32),
          pltpu.VMEM((half, _HISTP), jnp.int32),
          pltpu.VMEM((2, _RING, 2 * _PKW), jnp.uint32),
          pltpu.VMEM((rows_per_w, _PROJ), jnp.float32),
          pltpu.SemaphoreType.DMA,
          pltpu.SemaphoreType.DMA,
      ],
      compiler_params=pltpu.CompilerParams(use_tc_tiling_on_sc=True,
                                           needs_layout_passes=False),
  )
  def k(pair_hbm, hoff_hbm, proj_hbm, out_hbm, idx_v, hoff_v, rows_v, acc_v,
        sem0, sem1):
    wid = lax.axis_index("s") * mesh.num_cores + lax.axis_index("c")
    base = wid * rows_per_w

    starts = [min(16 * j, _HIST - _CHUNK) for j in range(_NCHUNK)]

    def fire(bh, par, sem):
      for j, s in enumerate(starts):
        iv = idx_v[bh, pl.ds(s, _CHUNK)]
        pltpu.async_copy(proj_hbm.at[iv],
                         rows_v.at[par, pl.ds(16 * j, _CHUNK)], sem)

    def drain(bh, par, sem):
      for j, s in enumerate(starts):
        iv = idx_v[bh, pl.ds(s, _CHUNK)]
        pltpu.make_async_copy(
            proj_hbm.at[iv], rows_v.at[par, pl.ds(16 * j, _CHUNK)],
            sem).wait()

    def acc_row(par, row, off, accs):
      for m in range(4):
        w = rows_v[par, row, pl.ds(off + 16 * m, 16)]
        ab = plsc.bitcast(w, jnp.bfloat16)
        a, b = plsc.unpack(ab, format=plsc.PackFormat.INTERLEAVED)
        accs[m] = accs[m] + a
        accs[4 + m] = accs[4 + m] + b
      return accs

    def accumulate_and_store(b, bh, par):
      accs = tuple(jnp.zeros((16,), jnp.float32) for _ in range(8))

      def body(kk, accs):
        accs = list(accs)
        hv = hoff_v[bh, pl.ds(kk * 16, 16)]
        for r in range(16):
          slot = kk * 16 + r
          ring_row = slot
          accs = acc_row(par, ring_row, hv[r], accs)
        return tuple(accs)

      accs = list(lax.fori_loop(0, 192 // 16, body, accs))
      hv = hoff_v[bh, pl.ds(_HIST - 16, 16)]
      for r in range(8, 16):
        accs = acc_row(par, 192 + r, hv[r], accs)

      for m in range(8):
        acc_v[b, pl.ds(m * 16, 16)] = accs[m]

    for h in range(2):
      hbase = base + h * half
      pltpu.sync_copy(pair_hbm.at[pl.ds(hbase, half)], idx_v)
      pltpu.sync_copy(hoff_hbm.at[pl.ds(hbase, half)], hoff_v)

      fire(0, 0, sem0)

      def step(bb, _, h=h):
        b0 = 2 * bb
        b1 = 2 * bb + 1
        fire(b1, 1, sem1)
        drain(b0, 0, sem0)
        accumulate_and_store(h * half + b0, b0, 0)

        @pl.when(bb < half // 2 - 1)
        def _():
          fire(b0 + 2, 0, sem0)

        drain(b1, 1, sem1)
        accumulate_and_store(h * half + b1, b1, 1)
        return 0

      lax.fori_loop(0, half // 2, step, 0)

    pltpu.sync_copy(acc_v, out_hbm.at[pl.ds(base, rows_per_w)])

  return k(pair2, hoff2, proj)


def _tc_finish(split2, fc_bias2):
  def body(x_ref, b_ref, o_ref):
    o_ref[:, :] = x_ref[:, :_NCLASS] * (1.0 / _HIST) + b_ref[:, :]

  return pl.pallas_call(
      body,
      out_shape=jax.ShapeDtypeStruct((_BATCH, _NCLASS), jnp.float32),
  )(split2, fc_bias2)


def kernel(text, embed_table, fc_weight, fc_bias):
  ti = text.astype(jnp.int32)
  pad = ((0, 0), (0, _HISTP - _HIST))
  blk = ti // _MM_BLK
  loc = ti % _MM_BLK
  pair2 = jnp.pad(blk * (_MM_BLK // 2) + (loc & (_MM_BLK // 2 - 1)), pad)
  hoff2 = jnp.pad((loc // (_MM_BLK // 2)) << 6, pad)
  proj = _tc_project(embed_table.T, fc_weight)
  split2 = _sc_gather_sum(pair2, hoff2, proj)
  return _tc_finish(split2, fc_bias.reshape(1, _NCLASS))


# speedup vs baseline: 1.4104x; 1.4104x over previous
import functools

import jax
import jax.numpy as jnp
from jax import lax
from jax.experimental import pallas as pl
from jax.experimental.pallas import tpu as pltpu
from jax.experimental.pallas import tpu_sc as plsc

_BATCH = 4096
_HIST = 200
_HISTP = 256
_RING = 208
_VOCAB = 1000000
_DIM = 64
_NCLASS = 100
_PROJ = 128

_NCHUNK = 13
_CHUNK = 16
_PGRP = _PROJ // 16

_MM_BLK = 2048


def _tc_project(table_t, fc_weight):
  def body(t_ref, w_ref, o_ref):
    acc = lax.dot_general(
        t_ref[:, :].astype(jnp.bfloat16), w_ref[:, :].astype(jnp.bfloat16),
        dimension_numbers=(((0,), (1,)), ((), ())),
        preferred_element_type=jnp.float32,
    )
    o_ref[:, :] = jnp.pad(acc, ((0, 0), (0, _PROJ - _NCLASS)))

  return pl.pallas_call(
      body,
      grid=((_VOCAB + _MM_BLK - 1) // _MM_BLK,),
      in_specs=[
          pl.BlockSpec((_DIM, _MM_BLK), lambda i: (0, i)),
          pl.BlockSpec((_NCLASS, _DIM), lambda i: (0, 0)),
      ],
      out_specs=pl.BlockSpec((_MM_BLK, _PROJ), lambda i: (i, 0)),
      out_shape=jax.ShapeDtypeStruct((_VOCAB, _PROJ), jnp.float32),
      compiler_params=pltpu.CompilerParams(fuse_transposed_lhs_in_matmul=True),
  )(table_t, fc_weight)


def _sc_gather_sum(idx2, proj):
  mesh = plsc.VectorSubcoreMesh(core_axis_name="c", subcore_axis_name="s")
  nw = mesh.num_cores * mesh.num_subcores
  rows_per_w = _BATCH // nw
  half = rows_per_w // 2

  @functools.partial(
      pl.kernel,
      out_type=jax.ShapeDtypeStruct((_BATCH, _PROJ), jnp.float32),
      mesh=mesh,
      scratch_types=[
          pltpu.VMEM((half, _HISTP), jnp.int32),
          pltpu.VMEM((2, _RING, _PROJ), jnp.float32),
          pltpu.VMEM((rows_per_w, _PROJ), jnp.float32),
          pltpu.SemaphoreType.DMA,
          pltpu.SemaphoreType.DMA,
      ],
      compiler_params=pltpu.CompilerParams(use_tc_tiling_on_sc=True),
  )
  def k(idx_hbm, proj_hbm, out_hbm, idx_v, rows_v, acc_v, sem0, sem1):
    wid = lax.axis_index("s") * mesh.num_cores + lax.axis_index("c")
    base = wid * rows_per_w

    starts = [min(16 * j, _HIST - _CHUNK) for j in range(_NCHUNK)]

    def fire(bh, par, sem):
      for j, s in enumerate(starts):
        iv = idx_v[bh, pl.ds(s, _CHUNK)]
        pltpu.async_copy(proj_hbm.at[iv],
                         rows_v.at[par, pl.ds(16 * j, _CHUNK)], sem)

    def drain(bh, par, sem):
      for j, s in enumerate(starts):
        iv = idx_v[bh, pl.ds(s, _CHUNK)]
        pltpu.make_async_copy(
            proj_hbm.at[iv], rows_v.at[par, pl.ds(16 * j, _CHUNK)],
            sem).wait()

    def acc_row(par, row, accs):
      for g in range(_PGRP):
        accs[g] = accs[g] + rows_v[par, row, pl.ds(g * 16, 16)]
      return accs

    def accumulate_and_store(b, par):
      accs = tuple(jnp.zeros((16,), jnp.float32) for _ in range(_PGRP))

      def body(kk, accs):
        accs = list(accs)
        for r in range(8):
          accs = acc_row(par, kk * 8 + r, accs)
        return tuple(accs)

      accs = list(lax.fori_loop(0, 192 // 8, body, accs))
      for row in range(200, 208):
        accs = acc_row(par, row, accs)

      for g in range(_PGRP):
        acc_v[b, pl.ds(g * 16, 16)] = accs[g]

    for h in range(2):
      hbase = base + h * half
      pltpu.sync_copy(idx_hbm.at[pl.ds(hbase, half)], idx_v)

      fire(0, 0, sem0)

      def step(bb, _, h=h):
        b0 = 2 * bb
        b1 = 2 * bb + 1
        fire(b1, 1, sem1)
        drain(b0, 0, sem0)
        accumulate_and_store(h * half + b0, 0)

        @pl.when(bb < half // 2 - 1)
        def _():
          fire(b0 + 2, 0, sem0)

        drain(b1, 1, sem1)
        accumulate_and_store(h * half + b1, 1)
        return 0

      lax.fori_loop(0, half // 2, step, 0)

    pltpu.sync_copy(acc_v, out_hbm.at[pl.ds(base, rows_per_w)])

  return k(idx2, proj)


def _tc_finish(sums2, fc_bias2):
  def body(x_ref, b_ref, o_ref):
    o_ref[:, :] = x_ref[:, :_NCLASS] * (1.0 / _HIST) + b_ref[:, :]

  return pl.pallas_call(
      body,
      out_shape=jax.ShapeDtypeStruct((_BATCH, _NCLASS), jnp.float32),
  )(sums2, fc_bias2)


def kernel(text, embed_table, fc_weight, fc_bias):
  idx2 = jnp.pad(text.astype(jnp.int32), ((0, 0), (0, _HISTP - _HIST)))
  proj = _tc_project(embed_table.T, fc_weight)
  sums2 = _sc_gather_sum(idx2, proj)
  return _tc_finish(sums2, fc_bias.reshape(1, _NCLASS))


# speedup vs baseline: 2.0429x; 1.4485x over previous
import functools

import jax
import jax.numpy as jnp
from jax import lax
from jax.experimental import pallas as pl
from jax.experimental.pallas import tpu as pltpu
from jax.experimental.pallas import tpu_sc as plsc

_BATCH = 4096
_HIST = 200
_HISTP = 256
_RING = 208
_VOCAB = 1000000
_DIM = 64
_NCLASS = 100
_PROJ = 128

_NCHUNK = 13
_CHUNK = 16
_PGRP = _PROJ // 16

_MM_BLK = 8192


def _tc_project(table_t, fc_weight):
  def body(t_ref, w_ref, o_ref):
    acc = lax.dot_general(
        t_ref[:, :].astype(jnp.bfloat16), w_ref[:, :].astype(jnp.bfloat16),
        dimension_numbers=(((0,), (1,)), ((), ())),
        preferred_element_type=jnp.float32,
    )
    o_ref[:, :] = jnp.pad(acc, ((0, 0), (0, _PROJ - _NCLASS)))

  return pl.pallas_call(
      body,
      grid=((_VOCAB + _MM_BLK - 1) // _MM_BLK,),
      in_specs=[
          pl.BlockSpec((_DIM, _MM_BLK), lambda i: (0, i)),
          pl.BlockSpec((_NCLASS, _DIM), lambda i: (0, 0)),
      ],
      out_specs=pl.BlockSpec((_MM_BLK, _PROJ), lambda i: (i, 0)),
      out_shape=jax.ShapeDtypeStruct((_VOCAB, _PROJ), jnp.float32),
      compiler_params=pltpu.CompilerParams(fuse_transposed_lhs_in_matmul=True),
  )(table_t, fc_weight)


def _sc_gather_sum(idx2, proj):
  mesh = plsc.VectorSubcoreMesh(core_axis_name="c", subcore_axis_name="s")
  nw = mesh.num_cores * mesh.num_subcores
  rows_per_w = _BATCH // nw
  half = rows_per_w // 2

  @functools.partial(
      pl.kernel,
      out_type=jax.ShapeDtypeStruct((_BATCH, _PROJ), jnp.float32),
      mesh=mesh,
      scratch_types=[
          pltpu.VMEM((half, _HISTP), jnp.int32),
          pltpu.VMEM((2, _RING, _PROJ), jnp.float32),
          pltpu.VMEM((rows_per_w, _PROJ), jnp.float32),
          pltpu.SemaphoreType.DMA,
          pltpu.SemaphoreType.DMA,
      ],
      compiler_params=pltpu.CompilerParams(use_tc_tiling_on_sc=True),
  )
  def k(idx_hbm, proj_hbm, out_hbm, idx_v, rows_v, acc_v, sem0, sem1):
    wid = lax.axis_index("s") * mesh.num_cores + lax.axis_index("c")
    base = wid * rows_per_w

    starts = [min(16 * j, _HIST - _CHUNK) for j in range(_NCHUNK)]

    def fire(bh, par, sem):
      for j, s in enumerate(starts):
        iv = idx_v[bh, pl.ds(s, _CHUNK)]
        pltpu.async_copy(proj_hbm.at[iv],
                         rows_v.at[par, pl.ds(16 * j, _CHUNK)], sem)

    def drain(bh, par, sem):
      for j, s in enumerate(starts):
        iv = idx_v[bh, pl.ds(s, _CHUNK)]
        pltpu.make_async_copy(
            proj_hbm.at[iv], rows_v.at[par, pl.ds(16 * j, _CHUNK)],
            sem).wait()

    def acc_row(par, row, accs):
      for g in range(_PGRP):
        accs[g] = accs[g] + rows_v[par, row, pl.ds(g * 16, 16)]
      return accs

    def accumulate_and_store(b, par):
      accs = tuple(jnp.zeros((16,), jnp.float32) for _ in range(_PGRP))

      def body(kk, accs):
        accs = list(accs)
        for r in range(8):
          accs = acc_row(par, kk * 8 + r, accs)
        return tuple(accs)

      accs = list(lax.fori_loop(0, 192 // 8, body, accs))
      for row in range(200, 208):
        accs = acc_row(par, row, accs)

      for g in range(_PGRP):
        acc_v[b, pl.ds(g * 16, 16)] = accs[g]

    for h in range(2):
      hbase = base + h * half
      pltpu.sync_copy(idx_hbm.at[pl.ds(hbase, half)], idx_v)

      fire(0, 0, sem0)

      def step(bb, _, h=h):
        b0 = 2 * bb
        b1 = 2 * bb + 1
        fire(b1, 1, sem1)
        drain(b0, 0, sem0)
        accumulate_and_store(h * half + b0, 0)

        @pl.when(bb < half // 2 - 1)
        def _():
          fire(b0 + 2, 0, sem0)

        drain(b1, 1, sem1)
        accumulate_and_store(h * half + b1, 1)
        return 0

      lax.fori_loop(0, half // 2, step, 0)

    pltpu.sync_copy(acc_v, out_hbm.at[pl.ds(base, rows_per_w)])

  return k(idx2, proj)


def _tc_finish(sums2, fc_bias2):
  def body(x_ref, b_ref, o_ref):
    o_ref[:, :] = x_ref[:, :_NCLASS] * (1.0 / _HIST) + b_ref[:, :]

  return pl.pallas_call(
      body,
      out_shape=jax.ShapeDtypeStruct((_BATCH, _NCLASS), jnp.float32),
  )(sums2, fc_bias2)


def kernel(text, embed_table, fc_weight, fc_bias):
  idx2 = jnp.pad(text.astype(jnp.int32), ((0, 0), (0, _HISTP - _HIST)))
  proj = _tc_project(embed_table.T, fc_weight)
  sums2 = _sc_gather_sum(idx2, proj)
  return _tc_finish(sums2, fc_bias.reshape(1, _NCLASS))


# speedup vs baseline: 2.1008x; 1.0283x over previous
import functools

import jax
import jax.numpy as jnp
from jax import lax
from jax.experimental import pallas as pl
from jax.experimental.pallas import tpu as pltpu
from jax.experimental.pallas import tpu_sc as plsc

_BATCH = 4096
_HIST = 200
_HISTP = 256
_RING = 208
_VOCAB = 1000000
_DIM = 64
_NCLASS = 100
_PROJ = 128

_NCHUNK = 13
_CHUNK = 16
_PGRP = _PROJ // 16

_MM_BLK = 16384


def _tc_project(table_t, fc_weight):
  def body(t_ref, w_ref, o_ref):
    acc = lax.dot_general(
        t_ref[:, :].astype(jnp.bfloat16), w_ref[:, :].astype(jnp.bfloat16),
        dimension_numbers=(((0,), (1,)), ((), ())),
        preferred_element_type=jnp.float32,
    )
    o_ref[:, :] = jnp.pad(acc, ((0, 0), (0, _PROJ - _NCLASS)))

  return pl.pallas_call(
      body,
      grid=((_VOCAB + _MM_BLK - 1) // _MM_BLK,),
      in_specs=[
          pl.BlockSpec((_DIM, _MM_BLK), lambda i: (0, i)),
          pl.BlockSpec((_NCLASS, _DIM), lambda i: (0, 0)),
      ],
      out_specs=pl.BlockSpec((_MM_BLK, _PROJ), lambda i: (i, 0)),
      out_shape=jax.ShapeDtypeStruct((_VOCAB, _PROJ), jnp.float32),
      compiler_params=pltpu.CompilerParams(fuse_transposed_lhs_in_matmul=True),
  )(table_t, fc_weight)


def _sc_gather_sum(idx2, proj):
  mesh = plsc.VectorSubcoreMesh(core_axis_name="c", subcore_axis_name="s")
  nw = mesh.num_cores * mesh.num_subcores
  rows_per_w = _BATCH // nw
  half = rows_per_w // 2

  @functools.partial(
      pl.kernel,
      out_type=jax.ShapeDtypeStruct((_BATCH, _PROJ), jnp.float32),
      mesh=mesh,
      scratch_types=[
          pltpu.VMEM((half, _HISTP), jnp.int32),
          pltpu.VMEM((2, _RING, _PROJ), jnp.float32),
          pltpu.VMEM((rows_per_w, _PROJ), jnp.float32),
          pltpu.SemaphoreType.DMA,
          pltpu.SemaphoreType.DMA,
      ],
      compiler_params=pltpu.CompilerParams(use_tc_tiling_on_sc=True),
  )
  def k(idx_hbm, proj_hbm, out_hbm, idx_v, rows_v, acc_v, sem0, sem1):
    wid = lax.axis_index("s") * mesh.num_cores + lax.axis_index("c")
    base = wid * rows_per_w

    starts = [min(16 * j, _HIST - _CHUNK) for j in range(_NCHUNK)]

    def fire(bh, par, sem):
      for j, s in enumerate(starts):
        iv = idx_v[bh, pl.ds(s, _CHUNK)]
        pltpu.async_copy(proj_hbm.at[iv],
                         rows_v.at[par, pl.ds(16 * j, _CHUNK)], sem)

    def drain(bh, par, sem):
      for j, s in enumerate(starts):
        iv = idx_v[bh, pl.ds(s, _CHUNK)]
        pltpu.make_async_copy(
            proj_hbm.at[iv], rows_v.at[par, pl.ds(16 * j, _CHUNK)],
            sem).wait()

    def acc_row(par, row, accs):
      for g in range(_PGRP):
        accs[g] = accs[g] + rows_v[par, row, pl.ds(g * 16, 16)]
      return accs

    def accumulate_and_store(b, par):
      accs = tuple(jnp.zeros((16,), jnp.float32) for _ in range(_PGRP))

      def body(kk, accs):
        accs = list(accs)
        for r in range(8):
          accs = acc_row(par, kk * 8 + r, accs)
        return tuple(accs)

      accs = list(lax.fori_loop(0, 192 // 8, body, accs))
      for row in range(200, 208):
        accs = acc_row(par, row, accs)

      for g in range(_PGRP):
        acc_v[b, pl.ds(g * 16, 16)] = accs[g]

    for h in range(2):
      hbase = base + h * half
      pltpu.sync_copy(idx_hbm.at[pl.ds(hbase, half)], idx_v)

      fire(0, 0, sem0)

      def step(bb, _, h=h):
        b0 = 2 * bb
        b1 = 2 * bb + 1
        fire(b1, 1, sem1)
        drain(b0, 0, sem0)
        accumulate_and_store(h * half + b0, 0)

        @pl.when(bb < half // 2 - 1)
        def _():
          fire(b0 + 2, 0, sem0)

        drain(b1, 1, sem1)
        accumulate_and_store(h * half + b1, 1)
        return 0

      lax.fori_loop(0, half // 2, step, 0)

    pltpu.sync_copy(acc_v, out_hbm.at[pl.ds(base, rows_per_w)])

  return k(idx2, proj)


def _tc_finish(sums2, fc_bias2):
  def body(x_ref, b_ref, o_ref):
    o_ref[:, :] = x_ref[:, :_NCLASS] * (1.0 / _HIST) + b_ref[:, :]

  return pl.pallas_call(
      body,
      out_shape=jax.ShapeDtypeStruct((_BATCH, _NCLASS), jnp.float32),
  )(sums2, fc_bias2)


def kernel(text, embed_table, fc_weight, fc_bias):
  idx2 = jnp.pad(text.astype(jnp.int32), ((0, 0), (0, _HISTP - _HIST)))
  proj = _tc_project(embed_table.T, fc_weight)
  sums2 = _sc_gather_sum(idx2, proj)
  return _tc_finish(sums2, fc_bias.reshape(1, _NCLASS))


# speedup vs baseline: 2.2060x; 1.0501x over previous
import functools

import jax
import jax.numpy as jnp
from jax import lax
from jax.experimental import pallas as pl
from jax.experimental.pallas import tpu as pltpu
from jax.experimental.pallas import tpu_sc as plsc

_BATCH = 4096
_HIST = 200
_HISTP = 256
_RING = 208
_VOCAB = 1000000
_DIM = 64
_NCLASS = 100
_PROJ = 128

_NCHUNK = 13
_CHUNK = 16
_PGRP = _PROJ // 16

_MM_BLK = 16384


def _tc_project(table_t, fc_weight):
  def body(t_ref, w_ref, o_ref):
    acc = lax.dot_general(
        t_ref[:, :].astype(jnp.bfloat16), w_ref[:, :].astype(jnp.bfloat16),
        dimension_numbers=(((0,), (1,)), ((), ())),
        preferred_element_type=jnp.float32,
    )
    o_ref[:, :] = jnp.pad(acc, ((0, 0), (0, _PROJ - _NCLASS)))

  return pl.pallas_call(
      body,
      grid=((_VOCAB + _MM_BLK - 1) // _MM_BLK,),
      in_specs=[
          pl.BlockSpec((_DIM, _MM_BLK), lambda i: (0, i)),
          pl.BlockSpec((_NCLASS, _DIM), lambda i: (0, 0)),
      ],
      out_specs=pl.BlockSpec((_MM_BLK, _PROJ), lambda i: (i, 0)),
      out_shape=jax.ShapeDtypeStruct((_VOCAB, _PROJ), jnp.float32),
      compiler_params=pltpu.CompilerParams(fuse_transposed_lhs_in_matmul=True),
  )(table_t, fc_weight)


def _sc_gather_sum(idx2, proj):
  mesh = plsc.VectorSubcoreMesh(core_axis_name="c", subcore_axis_name="s")
  nw = mesh.num_cores * mesh.num_subcores
  rows_per_w = _BATCH // nw
  qrt = rows_per_w // 4

  @functools.partial(
      pl.kernel,
      out_type=jax.ShapeDtypeStruct((_BATCH, _PROJ), jnp.float32),
      mesh=mesh,
      scratch_types=[
          pltpu.VMEM((qrt, _HISTP), jnp.int32),
          pltpu.VMEM((4, _RING, _PROJ), jnp.float32),
          pltpu.VMEM((qrt, _PROJ), jnp.float32),
          pltpu.SemaphoreType.DMA,
          pltpu.SemaphoreType.DMA,
          pltpu.SemaphoreType.DMA,
          pltpu.SemaphoreType.DMA,
      ],
      compiler_params=pltpu.CompilerParams(use_tc_tiling_on_sc=True),
  )
  def k(idx_hbm, proj_hbm, out_hbm, idx_v, rows_v, acc_v, s0, s1, s2, s3):
    wid = lax.axis_index("s") * mesh.num_cores + lax.axis_index("c")
    base = wid * rows_per_w

    starts = [min(16 * j, _HIST - _CHUNK) for j in range(_NCHUNK)]

    def fire(bh, par, sem):
      for j, s in enumerate(starts):
        iv = idx_v[bh, pl.ds(s, _CHUNK)]
        pltpu.async_copy(proj_hbm.at[iv],
                         rows_v.at[par, pl.ds(16 * j, _CHUNK)], sem)

    def drain(bh, par, sem):
      for j, s in enumerate(starts):
        iv = idx_v[bh, pl.ds(s, _CHUNK)]
        pltpu.make_async_copy(
            proj_hbm.at[iv], rows_v.at[par, pl.ds(16 * j, _CHUNK)],
            sem).wait()

    def acc_row(par, row, accs):
      for g in range(_PGRP):
        accs[g] = accs[g] + rows_v[par, row, pl.ds(g * 16, 16)]
      return accs

    def accumulate_and_store(b, par):
      accs = tuple(jnp.zeros((16,), jnp.float32) for _ in range(_PGRP))

      def body(kk, accs):
        accs = list(accs)
        for r in range(8):
          accs = acc_row(par, kk * 8 + r, accs)
        return tuple(accs)

      accs = list(lax.fori_loop(0, 192 // 8, body, accs))
      for row in range(200, 208):
        accs = acc_row(par, row, accs)

      for g in range(_PGRP):
        acc_v[b, pl.ds(g * 16, 16)] = accs[g]

    sems = (s0, s1, s2, s3)
    for q in range(4):
      qbase = base + q * qrt
      pltpu.sync_copy(idx_hbm.at[pl.ds(qbase, qrt)], idx_v)

      for r in range(3):
        fire(r, r, sems[r])

      def step(bb, _):
        for r in range(4):
          row = 4 * bb + r
          drain(row, r, sems[r])
          accumulate_and_store(row, r)

          @pl.when(row < qrt - 3)
          def _(row=row, r=r):
            fire(row + 3, (r + 3) % 4, sems[(r + 3) % 4])
        return 0

      lax.fori_loop(0, qrt // 4, step, 0)
      pltpu.sync_copy(acc_v, out_hbm.at[pl.ds(qbase, qrt)])

  return k(idx2, proj)


def _tc_finish(sums2, fc_bias2):
  def body(x_ref, b_ref, o_ref):
    o_ref[:, :] = x_ref[:, :_NCLASS] * (1.0 / _HIST) + b_ref[:, :]

  return pl.pallas_call(
      body,
      out_shape=jax.ShapeDtypeStruct((_BATCH, _NCLASS), jnp.float32),
  )(sums2, fc_bias2)


def kernel(text, embed_table, fc_weight, fc_bias):
  idx2 = jnp.pad(text.astype(jnp.int32), ((0, 0), (0, _HISTP - _HIST)))
  proj = _tc_project(embed_table.T, fc_weight)
  sums2 = _sc_gather_sum(idx2, proj)
  return _tc_finish(sums2, fc_bias.reshape(1, _NCLASS))
